# baseline (speedup 1.0000x reference)
import functools

import jax
import jax.numpy as jnp
from jax import lax
from jax.experimental import pallas as pl
from jax.experimental.pallas import tpu as pltpu
from jax.experimental.pallas import tpu_sc as plsc

_N = 10000
_D = 128
_E = 320000


_NCORES = 2
_NSUB = 16
_WORKERS = _NCORES * _NSUB
_CHUNK = 104
_EPW = _E // _WORKERS
_NCH = 99
_EPWP = _NCH * _CHUNK
_TRIPS = _NCH // 3
_NJUNK = 8
_RPT = 624
_ZROWS = 208

@functools.cache
def _get_sc_segsum():
    mesh = plsc.VectorSubcoreMesh(core_axis_name="c", subcore_axis_name="s",
                                  num_cores=_NCORES, num_subcores=_NSUB)

    @functools.partial(
        pl.kernel,
        mesh=mesh,
        out_type=[
            jax.ShapeDtypeStruct((_N, _D), jnp.float32),
            jax.ShapeDtypeStruct((_N, _D), jnp.float32),
        ],
        scratch_types=[
            pltpu.VMEM((8, _CHUNK), jnp.int32),
            pltpu.VMEM((8, _CHUNK), jnp.int32),
            pltpu.VMEM((8, _CHUNK), jnp.int32),
            pltpu.VMEM((8, _CHUNK), jnp.int32),
            pltpu.VMEM((8, _CHUNK), jnp.int32),
            pltpu.VMEM((8, _CHUNK), jnp.int32),
            pltpu.VMEM((8, _CHUNK), jnp.int32),
            pltpu.VMEM((8, _CHUNK), jnp.int32),
            pltpu.VMEM((8, _CHUNK), jnp.int32),
            pltpu.VMEM((_CHUNK, _D), jnp.float32),
            pltpu.VMEM((_CHUNK, _D), jnp.float32),
            pltpu.VMEM((_CHUNK, _D), jnp.float32),
            pltpu.VMEM_SHARED((_N + _NJUNK, _D), jnp.float32),
            pltpu.SemaphoreType.DMA,
            pltpu.SemaphoreType.DMA,
            pltpu.SemaphoreType.DMA,
            pltpu.SemaphoreType.DMA,
            pltpu.SemaphoreType.DMA,
            pltpu.SemaphoreType.DMA,
            pltpu.SemaphoreType.DMA,
            pltpu.SemaphoreType.DMA,
            pltpu.SemaphoreType.DMA,
        ],
    )
    def _sc_segsum(h_hbm, edges_hbm, zeros_hbm, agg0_hbm, agg1_hbm,
                   pck0, pck1, pck2, sidx0, didx0, sidx1, didx1,
                   sidx2, didx2, rows0, rows1, rows2, acc_sh,
                   psem0, psem1, psem2, gsem0, gsem1, gsem2,
                   ssem0, ssem1, ssem2):
        cid = lax.axis_index("c")
        sid = lax.axis_index("s")
        wid = cid * _NSUB + sid

        pck = (pck0, pck1, pck2)
        sidx = (sidx0, sidx1, sidx2)
        didx = (didx0, didx1, didx2)
        rows = (rows0, rows1, rows2)
        psem = (psem0, psem1, psem2)
        gsem = (gsem0, gsem1, gsem2)
        ssem = (ssem0, ssem1, ssem2)

        def stage_packed(c, r):
            pltpu.async_copy(
                edges_hbm.at[pl.ds((wid * _NCH + c) * _CHUNK, _CHUNK)],
                pck[r].at[0], psem[r])

        def wait_packed(c, r):
            pltpu.make_async_copy(
                edges_hbm.at[pl.ds((wid * _NCH + c) * _CHUNK, _CHUNK)],
                pck[r].at[0], psem[r]).wait()

        def unpack(r):
            offs = [16 * c for c in range(_CHUNK // 16)] + [_CHUNK - 16]
            for o in offs:
                v = pck[r][0, pl.ds(o, 16)]
                sidx[r][0, pl.ds(o, 16)] = v & 0xFFFF
                didx[r][0, pl.ds(o, 16)] = lax.shift_right_logical(v, 16)

        def gather(r):
            pltpu.async_copy(h_hbm.at[sidx[r].at[0]], rows[r], gsem[r])

        def wait_gather(r):
            pltpu.make_async_copy(h_hbm.at[sidx[r].at[0]], rows[r],
                                  gsem[r]).wait()

        def scatter(r):
            pltpu.async_copy(rows[r], acc_sh.at[didx[r].at[0]], ssem[r],
                             add=True)

        def wait_scatter(r):
            pltpu.make_async_copy(rows[r], acc_sh.at[didx[r].at[0]],
                                  ssem[r]).wait()

        stage_packed(0, 0)
        stage_packed(1, 1)

        for k in range(_RPT // _ZROWS):
            pltpu.sync_copy(zeros_hbm,
                            acc_sh.at[pl.ds(sid * _RPT + k * _ZROWS, _ZROWS)])

        @pl.when(sid == _NSUB - 1)
        def _():
            pltpu.sync_copy(zeros_hbm.at[pl.ds(0, 16)],
                            acc_sh.at[pl.ds(_NSUB * _RPT, 16)])

        plsc.subcore_barrier()

        def body(t, carry):
            for k in range(3):
                c = 3 * t + k
                r = k

                @pl.when(t > 0)
                def _():
                    wait_scatter(r)

                wait_packed(c, r)
                unpack(r)
                gather(r)

                @pl.when(c + 2 < _NCH)
                def _():
                    stage_packed(c + 2, (r + 2) % 3)

                rp = (r + 1) % 3

                @pl.when(c >= 2)
                def _():
                    wait_gather(rp)
                    scatter(rp)

            return carry

        lax.fori_loop(0, _TRIPS, body, 0)
        wait_gather(1)
        scatter(1)
        wait_gather(2)
        scatter(2)
        wait_scatter(0)
        wait_scatter(1)
        wait_scatter(2)
        plsc.subcore_barrier()

        @pl.when(cid == 0)
        def _():
            pltpu.sync_copy(acc_sh.at[pl.ds(sid * _RPT, _RPT)],
                            agg0_hbm.at[pl.ds(sid * _RPT, _RPT)])

            @pl.when(sid == _NSUB - 1)
            def _():
                pltpu.sync_copy(acc_sh.at[pl.ds(_NSUB * _RPT, 16)],
                                agg0_hbm.at[pl.ds(_NSUB * _RPT, 16)])

        @pl.when(cid == 1)
        def _():
            pltpu.sync_copy(acc_sh.at[pl.ds(sid * _RPT, _RPT)],
                            agg1_hbm.at[pl.ds(sid * _RPT, _RPT)])

            @pl.when(sid == _NSUB - 1)
            def _():
                pltpu.sync_copy(acc_sh.at[pl.ds(_NSUB * _RPT, 16)],
                                agg1_hbm.at[pl.ds(_NSUB * _RPT, 16)])

    return _sc_segsum


_BLK = 2000
_G = _N // _BLK


def _phase0(h_ref, a0_ref, a1_ref, w1t_ref, b1_ref, y_scr, st_scr, i):
    z = h_ref[...] + a0_ref[...] + a1_ref[...]
    y = jnp.dot(z, w1t_ref[...], preferred_element_type=jnp.float32)
    y = jnp.maximum(y + b1_ref[...], 0.0)
    y_scr[pl.ds(i * _BLK, _BLK), :] = y

    @pl.when(i == 0)
    def _():
        st_scr[...] = jnp.zeros_like(st_scr)

    s = jnp.sum(y, axis=0, keepdims=True)
    q = jnp.sum(y * y, axis=0, keepdims=True)
    st_scr[...] += jnp.concatenate(
        [s, q, jnp.zeros((6, _D), jnp.float32)], axis=0)


def _phase1_norm(g_ref, be_ref, w2t_ref, b2_ref, y_scr, st_scr, j):
    st = st_scr[...]
    mean = st[0:1, :] * (1.0 / _N)
    var = st[1:2, :] * (1.0 / _N) - mean * mean
    scale = g_ref[...] * lax.rsqrt(var + 1e-5)
    shift = be_ref[...] - mean * scale
    yn = y_scr[pl.ds(j * _BLK, _BLK), :] * scale + shift
    o = jnp.dot(yn, w2t_ref[...], preferred_element_type=jnp.float32)
    return jnp.maximum(o + b2_ref[...], 0.0)


def _layer_body(h_ref, a0_ref, a1_ref, w1t_ref, b1_ref, g_ref, be_ref,
                w2t_ref, b2_ref, o_ref, y_scr, st_scr):
    i = pl.program_id(0)

    @pl.when(i < _G)
    def _():
        _phase0(h_ref, a0_ref, a1_ref, w1t_ref, b1_ref, y_scr, st_scr, i)

    @pl.when(i >= _G)
    def _():
        o_ref[...] = _phase1_norm(g_ref, be_ref, w2t_ref, b2_ref,
                                  y_scr, st_scr, i - _G)


def _layer_res_body(h_ref, a0_ref, a1_ref, w1t_ref, b1_ref, g_ref, be_ref,
                    w2t_ref, b2_ref, r_ref, o_ref, y_scr, st_scr):
    i = pl.program_id(0)

    @pl.when(i < _G)
    def _():
        _phase0(h_ref, a0_ref, a1_ref, w1t_ref, b1_ref, y_scr, st_scr, i)

    @pl.when(i >= _G)
    def _():
        o_ref[...] = (_phase1_norm(g_ref, be_ref, w2t_ref, b2_ref,
                                   y_scr, st_scr, i - _G) + r_ref[...])


def _layer_head_body(h_ref, a0_ref, a1_ref, w1t_ref, b1_ref, g_ref, be_ref,
                     w2t_ref, b2_ref, r_ref, f1t_ref, f1b_ref, f2t_ref,
                     f2b_ref, o_ref, y_scr, st_scr):
    i = pl.program_id(0)

    @pl.when(i < _G)
    def _():
        _phase0(h_ref, a0_ref, a1_ref, w1t_ref, b1_ref, y_scr, st_scr, i)

    @pl.when(i >= _G)
    def _():
        h5 = (_phase1_norm(g_ref, be_ref, w2t_ref, b2_ref,
                           y_scr, st_scr, i - _G) + r_ref[...])
        t = jnp.dot(h5, f1t_ref[...], preferred_element_type=jnp.float32)
        t = h5 + jnp.maximum(t + f1b_ref[...], 0.0)
        o = jnp.dot(t, f2t_ref[...], preferred_element_type=jnp.float32)
        o_ref[...] = o + f2b_ref[...]


def _p0_map(i):
    return (jnp.minimum(i, _G - 1), 0)


def _p1_map(i):
    return (jnp.maximum(i - _G, 0), 0)


def _const_map(i):
    return (0, 0)


_layer_specs = [
    pl.BlockSpec((_BLK, _D), _p0_map),
    pl.BlockSpec((_BLK, _D), _p0_map),
    pl.BlockSpec((_BLK, _D), _p0_map),
    pl.BlockSpec((_D, _D), _const_map),
    pl.BlockSpec((1, _D), _const_map),
    pl.BlockSpec((1, _D), _const_map),
    pl.BlockSpec((1, _D), _const_map),
    pl.BlockSpec((_D, _D), _const_map),
    pl.BlockSpec((1, _D), _const_map),
]

_layer_scratch = [
    pltpu.VMEM((_N, _D), jnp.float32),
    pltpu.VMEM((8, _D), jnp.float32),
]

_layer = pl.pallas_call(
    _layer_body,
    grid=(2 * _G,),
    in_specs=_layer_specs,
    out_specs=pl.BlockSpec((_BLK, _D), _p1_map),
    out_shape=jax.ShapeDtypeStruct((_N, _D), jnp.float32),
    scratch_shapes=_layer_scratch,
)

_layer_res = pl.pallas_call(
    _layer_res_body,
    grid=(2 * _G,),
    in_specs=_layer_specs + [pl.BlockSpec((_BLK, _D), _p1_map)],
    out_specs=pl.BlockSpec((_BLK, _D), _p1_map),
    out_shape=jax.ShapeDtypeStruct((_N, _D), jnp.float32),
    scratch_shapes=_layer_scratch,
)

_layer_head = pl.pallas_call(
    _layer_head_body,
    grid=(2 * _G,),
    in_specs=_layer_specs + [
        pl.BlockSpec((_BLK, _D), _p1_map),
        pl.BlockSpec((_D, _D), _const_map),
        pl.BlockSpec((1, _D), _const_map),
        pl.BlockSpec((_D, 1), _const_map),
        pl.BlockSpec((1, 1), _const_map),
    ],
    out_specs=pl.BlockSpec((_BLK, 1), _p1_map),
    out_shape=jax.ShapeDtypeStruct((_N, 1), jnp.float32),
    scratch_shapes=_layer_scratch,
)


def kernel(x, edge_index, W1s, b1s, gammas, betas, W2s, b2s, fc1_w, fc1_b,
           fc2_w, fc2_b):
    packed = (edge_index[0] | (edge_index[1] << 16)).reshape(_WORKERS, _EPW)
    packed = jnp.pad(packed, ((0, 0), (0, _EPWP - _EPW)),
                     constant_values=_N << 16)
    packed = packed.reshape(-1)
    zeros = jnp.zeros((_ZROWS, _D), jnp.float32)
    W1ts = jnp.swapaxes(W1s, 1, 2)
    W2ts = jnp.swapaxes(W2s, 1, 2)

    sc_segsum = _get_sc_segsum()
    h = x
    x0 = x
    for i in range(5):
        agg0, agg1 = sc_segsum(h, packed, zeros)
        args = (h, agg0, agg1, W1ts[i], b1s[i][None], gammas[i][None],
                betas[i][None], W2ts[i], b2s[i][None])
        if i % 2 == 1:
            h = _layer_res(*args, x0)
            x0 = h
        else:
            h = _layer(*args)
    agg0, agg1 = sc_segsum(h, packed, zeros)
    return _layer_head(h, agg0, agg1, W1ts[5], b1s[5][None], gammas[5][None],
                       betas[5][None], W2ts[5], b2s[5][None], x0,
                       fc1_w.T, fc1_b[None], fc2_w.T, fc2_b[None])


# speedup vs baseline: 2.8044x; 2.8044x over previous
import functools

import jax
import jax.numpy as jnp
from jax import lax
from jax.experimental import pallas as pl
from jax.experimental.pallas import tpu as pltpu
from jax.experimental.pallas import tpu_sc as plsc

_N = 10000
_D = 128
_E = 320000


_NCORES = 2
_NSUB = 16
_WORKERS = _NCORES * _NSUB
_CHUNK = 128
_EPW = _E // _WORKERS
_NFULL = _EPW // _CHUNK
_TAIL = _EPW - _NFULL * _CHUNK
_NCH = _NFULL + 1
_HALF = _NFULL // 2
_RPT = 624
_ZROWS = 208

@functools.cache
def _get_sc_segsum():
    mesh = plsc.VectorSubcoreMesh(core_axis_name="c", subcore_axis_name="s",
                                  num_cores=_NCORES, num_subcores=_NSUB)

    @functools.partial(
        pl.kernel,
        mesh=mesh,
        out_type=[
            jax.ShapeDtypeStruct((_N, _D), jnp.float32),
            jax.ShapeDtypeStruct((_N, _D), jnp.float32),
        ],
        scratch_types=[
            pltpu.VMEM((_NCH, _CHUNK), jnp.int32),
            pltpu.VMEM((8, _CHUNK), jnp.int32),
            pltpu.VMEM((8, _CHUNK), jnp.int32),
            pltpu.VMEM((8, _CHUNK), jnp.int32),
            pltpu.VMEM((8, _CHUNK), jnp.int32),
            pltpu.VMEM((8, _TAIL), jnp.int32),
            pltpu.VMEM((8, _TAIL), jnp.int32),
            pltpu.VMEM((_CHUNK, _D), jnp.float32),
            pltpu.VMEM((_CHUNK, _D), jnp.float32),
            pltpu.VMEM_SHARED((_N, _D), jnp.float32),
            pltpu.SemaphoreType.DMA,
            pltpu.SemaphoreType.DMA,
            pltpu.SemaphoreType.DMA,
            pltpu.SemaphoreType.DMA,
        ],
    )
    def _sc_segsum(h_hbm, edges_hbm, zeros_hbm, agg0_hbm, agg1_hbm,
                   packed_v, sidx_a, didx_a, sidx_b, didx_b,
                   sidx_e, didx_e, rows_a, rows_b, acc_sh,
                   gsem_a, gsem_b, ssem_a, ssem_b):
        cid = lax.axis_index("c")
        sid = lax.axis_index("s")
        wid = cid * _NSUB + sid

        pltpu.async_copy(edges_hbm.at[wid], packed_v, gsem_b)
        for k in range(_RPT // _ZROWS):
            pltpu.async_copy(
                zeros_hbm, acc_sh.at[pl.ds(sid * _RPT + k * _ZROWS, _ZROWS)],
                gsem_a)

        @pl.when(sid == _NSUB - 1)
        def _():
            pltpu.sync_copy(zeros_hbm.at[pl.ds(0, 16)],
                            acc_sh.at[pl.ds(_NSUB * _RPT, 16)])

        for k in range(_RPT // _ZROWS):
            pltpu.make_async_copy(
                zeros_hbm, acc_sh.at[pl.ds(sid * _RPT + k * _ZROWS, _ZROWS)],
                gsem_a).wait()
        pltpu.make_async_copy(edges_hbm.at[wid], packed_v, gsem_b).wait()
        plsc.subcore_barrier()

        def unpack(j, sidx, didx):
            for c in range(_CHUNK // 16):
                v = packed_v[j, pl.ds(c * 16, 16)]
                sidx[0, pl.ds(c * 16, 16)] = v & 0xFFFF
                didx[0, pl.ds(c * 16, 16)] = lax.shift_right_logical(v, 16)

        unpack(0, sidx_a, didx_a)
        pltpu.async_copy(h_hbm.at[sidx_a.at[0]], rows_a, gsem_a)

        def body(i, carry):
            c0 = 2 * i
            c1 = c0 + 1
            unpack(c1, sidx_b, didx_b)
            pltpu.async_copy(h_hbm.at[sidx_b.at[0]], rows_b, gsem_b)
            pltpu.make_async_copy(h_hbm.at[sidx_a.at[0]], rows_a,
                                  gsem_a).wait()
            pltpu.async_copy(rows_a, acc_sh.at[didx_a.at[0]], ssem_a,
                             add=True)
            pltpu.make_async_copy(h_hbm.at[sidx_b.at[0]], rows_b,
                                  gsem_b).wait()
            pltpu.async_copy(rows_b, acc_sh.at[didx_b.at[0]], ssem_b,
                             add=True)
            pltpu.make_async_copy(rows_a, acc_sh.at[didx_a.at[0]],
                                  ssem_a).wait()

            @pl.when(i < _HALF - 1)
            def _():
                unpack(c0 + 2, sidx_a, didx_a)
                pltpu.async_copy(h_hbm.at[sidx_a.at[0]], rows_a, gsem_a)

            pltpu.make_async_copy(rows_b, acc_sh.at[didx_b.at[0]],
                                  ssem_b).wait()
            return carry

        lax.fori_loop(0, _HALF, body, 0)
        v = packed_v[_NCH - 1, pl.ds(0, _TAIL)]
        sidx_e[0, pl.ds(0, _TAIL)] = v & 0xFFFF
        didx_e[0, pl.ds(0, _TAIL)] = lax.shift_right_logical(v, 16)
        pltpu.async_copy(h_hbm.at[sidx_e.at[0]], rows_b.at[pl.ds(0, _TAIL)],
                         gsem_b)
        pltpu.make_async_copy(h_hbm.at[sidx_e.at[0]],
                              rows_b.at[pl.ds(0, _TAIL)], gsem_b).wait()
        pltpu.sync_copy(rows_b.at[pl.ds(0, _TAIL)],
                        acc_sh.at[didx_e.at[0]], add=True)
        plsc.subcore_barrier()

        @pl.when(cid == 0)
        def _():
            pltpu.sync_copy(acc_sh.at[pl.ds(sid * _RPT, _RPT)],
                            agg0_hbm.at[pl.ds(sid * _RPT, _RPT)])

            @pl.when(sid == _NSUB - 1)
            def _():
                pltpu.sync_copy(acc_sh.at[pl.ds(_NSUB * _RPT, 16)],
                                agg0_hbm.at[pl.ds(_NSUB * _RPT, 16)])

        @pl.when(cid == 1)
        def _():
            pltpu.sync_copy(acc_sh.at[pl.ds(sid * _RPT, _RPT)],
                            agg1_hbm.at[pl.ds(sid * _RPT, _RPT)])

            @pl.when(sid == _NSUB - 1)
            def _():
                pltpu.sync_copy(acc_sh.at[pl.ds(_NSUB * _RPT, 16)],
                                agg1_hbm.at[pl.ds(_NSUB * _RPT, 16)])

    return _sc_segsum


_BLK = 2000
_G = _N // _BLK


def _phase0(h_ref, a0_ref, a1_ref, w1t_ref, b1_ref, y_scr, st_scr, i):
    z = h_ref[...] + a0_ref[...] + a1_ref[...]
    y = jnp.dot(z, w1t_ref[...], preferred_element_type=jnp.float32)
    y = jnp.maximum(y + b1_ref[...], 0.0)
    y_scr[pl.ds(i * _BLK, _BLK), :] = y

    @pl.when(i == 0)
    def _():
        st_scr[...] = jnp.zeros_like(st_scr)

    s = jnp.sum(y, axis=0, keepdims=True)
    q = jnp.sum(y * y, axis=0, keepdims=True)
    st_scr[...] += jnp.concatenate(
        [s, q, jnp.zeros((6, _D), jnp.float32)], axis=0)


def _phase1_norm(g_ref, be_ref, w2t_ref, b2_ref, y_scr, st_scr, j):
    st = st_scr[...]
    mean = st[0:1, :] * (1.0 / _N)
    var = st[1:2, :] * (1.0 / _N) - mean * mean
    scale = g_ref[...] * lax.rsqrt(var + 1e-5)
    shift = be_ref[...] - mean * scale
    yn = y_scr[pl.ds(j * _BLK, _BLK), :] * scale + shift
    o = jnp.dot(yn, w2t_ref[...], preferred_element_type=jnp.float32)
    return jnp.maximum(o + b2_ref[...], 0.0)


def _layer_body(h_ref, a0_ref, a1_ref, w1t_ref, b1_ref, g_ref, be_ref,
                w2t_ref, b2_ref, o_ref, y_scr, st_scr):
    i = pl.program_id(0)

    @pl.when(i < _G)
    def _():
        _phase0(h_ref, a0_ref, a1_ref, w1t_ref, b1_ref, y_scr, st_scr, i)

    @pl.when(i >= _G)
    def _():
        o_ref[...] = _phase1_norm(g_ref, be_ref, w2t_ref, b2_ref,
                                  y_scr, st_scr, i - _G)


def _layer_res_body(h_ref, a0_ref, a1_ref, w1t_ref, b1_ref, g_ref, be_ref,
                    w2t_ref, b2_ref, r_ref, o_ref, y_scr, st_scr):
    i = pl.program_id(0)

    @pl.when(i < _G)
    def _():
        _phase0(h_ref, a0_ref, a1_ref, w1t_ref, b1_ref, y_scr, st_scr, i)

    @pl.when(i >= _G)
    def _():
        o_ref[...] = (_phase1_norm(g_ref, be_ref, w2t_ref, b2_ref,
                                   y_scr, st_scr, i - _G) + r_ref[...])


def _layer_head_body(h_ref, a0_ref, a1_ref, w1t_ref, b1_ref, g_ref, be_ref,
                     w2t_ref, b2_ref, r_ref, f1t_ref, f1b_ref, f2t_ref,
                     f2b_ref, o_ref, y_scr, st_scr):
    i = pl.program_id(0)

    @pl.when(i < _G)
    def _():
        _phase0(h_ref, a0_ref, a1_ref, w1t_ref, b1_ref, y_scr, st_scr, i)

    @pl.when(i >= _G)
    def _():
        h5 = (_phase1_norm(g_ref, be_ref, w2t_ref, b2_ref,
                           y_scr, st_scr, i - _G) + r_ref[...])
        t = jnp.dot(h5, f1t_ref[...], preferred_element_type=jnp.float32)
        t = h5 + jnp.maximum(t + f1b_ref[...], 0.0)
        o = jnp.dot(t, f2t_ref[...], preferred_element_type=jnp.float32)
        o_ref[...] = o + f2b_ref[...]


def _p0_map(i):
    return (jnp.minimum(i, _G - 1), 0)


def _p1_map(i):
    return (jnp.maximum(i - _G, 0), 0)


def _const_map(i):
    return (0, 0)


_layer_specs = [
    pl.BlockSpec((_BLK, _D), _p0_map),
    pl.BlockSpec((_BLK, _D), _p0_map),
    pl.BlockSpec((_BLK, _D), _p0_map),
    pl.BlockSpec((_D, _D), _const_map),
    pl.BlockSpec((1, _D), _const_map),
    pl.BlockSpec((1, _D), _const_map),
    pl.BlockSpec((1, _D), _const_map),
    pl.BlockSpec((_D, _D), _const_map),
    pl.BlockSpec((1, _D), _const_map),
]

_layer_scratch = [
    pltpu.VMEM((_N, _D), jnp.float32),
    pltpu.VMEM((8, _D), jnp.float32),
]

_layer = pl.pallas_call(
    _layer_body,
    grid=(2 * _G,),
    in_specs=_layer_specs,
    out_specs=pl.BlockSpec((_BLK, _D), _p1_map),
    out_shape=jax.ShapeDtypeStruct((_N, _D), jnp.float32),
    scratch_shapes=_layer_scratch,
)

_layer_res = pl.pallas_call(
    _layer_res_body,
    grid=(2 * _G,),
    in_specs=_layer_specs + [pl.BlockSpec((_BLK, _D), _p1_map)],
    out_specs=pl.BlockSpec((_BLK, _D), _p1_map),
    out_shape=jax.ShapeDtypeStruct((_N, _D), jnp.float32),
    scratch_shapes=_layer_scratch,
)

_layer_head = pl.pallas_call(
    _layer_head_body,
    grid=(2 * _G,),
    in_specs=_layer_specs + [
        pl.BlockSpec((_BLK, _D), _p1_map),
        pl.BlockSpec((_D, _D), _const_map),
        pl.BlockSpec((1, _D), _const_map),
        pl.BlockSpec((_D, 1), _const_map),
        pl.BlockSpec((1, 1), _const_map),
    ],
    out_specs=pl.BlockSpec((_BLK, 1), _p1_map),
    out_shape=jax.ShapeDtypeStruct((_N, 1), jnp.float32),
    scratch_shapes=_layer_scratch,
)


def kernel(x, edge_index, W1s, b1s, gammas, betas, W2s, b2s, fc1_w, fc1_b,
           fc2_w, fc2_b):
    packed = (edge_index[0] | (edge_index[1] << 16)).reshape(_WORKERS, _EPW)
    packed = jnp.pad(packed, ((0, 0), (0, _NCH * _CHUNK - _EPW)))
    packed = packed.reshape(_WORKERS, _NCH, _CHUNK)
    zeros = jnp.zeros((_ZROWS, _D), jnp.float32)
    W1ts = jnp.swapaxes(W1s, 1, 2)
    W2ts = jnp.swapaxes(W2s, 1, 2)

    sc_segsum = _get_sc_segsum()
    h = x
    x0 = x
    for i in range(5):
        agg0, agg1 = sc_segsum(h, packed, zeros)
        args = (h, agg0, agg1, W1ts[i], b1s[i][None], gammas[i][None],
                betas[i][None], W2ts[i], b2s[i][None])
        if i % 2 == 1:
            h = _layer_res(*args, x0)
            x0 = h
        else:
            h = _layer(*args)
    agg0, agg1 = sc_segsum(h, packed, zeros)
    return _layer_head(h, agg0, agg1, W1ts[5], b1s[5][None], gammas[5][None],
                       betas[5][None], W2ts[5], b2s[5][None], x0,
                       fc1_w.T, fc1_b[None], fc2_w.T, fc2_b[None])


# speedup vs baseline: 3.4943x; 1.2460x over previous
import functools

import jax
import jax.numpy as jnp
from jax import lax
from jax.experimental import pallas as pl
from jax.experimental.pallas import tpu as pltpu
from jax.experimental.pallas import tpu_sc as plsc

_N = 10000
_D = 128
_E = 320000


_NCORES = 2
_NSUB = 16
_WORKERS = _NCORES * _NSUB
_CHUNK = 128
_EPW = _E // _WORKERS
_NFULL = _EPW // _CHUNK
_TAIL = _EPW - _NFULL * _CHUNK
_NCH = _NFULL + 1
_HALF = _NFULL // 2
_RPT = 624
_ZROWS = 208

@functools.cache
def _get_sc_segsum():
    mesh = plsc.VectorSubcoreMesh(core_axis_name="c", subcore_axis_name="s",
                                  num_cores=_NCORES, num_subcores=_NSUB)

    @functools.partial(
        pl.kernel,
        mesh=mesh,
        out_type=[
            jax.ShapeDtypeStruct((_N, _D), jnp.float32),
            jax.ShapeDtypeStruct((_N, _D), jnp.float32),
        ],
        scratch_types=[
            pltpu.VMEM((_NCH, _CHUNK), jnp.int32),
            pltpu.VMEM((8, _CHUNK), jnp.int32),
            pltpu.VMEM((8, _CHUNK), jnp.int32),
            pltpu.VMEM((8, _CHUNK), jnp.int32),
            pltpu.VMEM((8, _CHUNK), jnp.int32),
            pltpu.VMEM((8, _TAIL), jnp.int32),
            pltpu.VMEM((8, _TAIL), jnp.int32),
            pltpu.VMEM((_CHUNK, _D), jnp.float32),
            pltpu.VMEM((_CHUNK, _D), jnp.float32),
            pltpu.VMEM_SHARED((_N, _D), jnp.float32),
            pltpu.SemaphoreType.DMA,
            pltpu.SemaphoreType.DMA,
            pltpu.SemaphoreType.DMA,
            pltpu.SemaphoreType.DMA,
        ],
    )
    def _sc_segsum(h_hbm, edges_hbm, zeros_hbm, agg0_hbm, agg1_hbm,
                   packed_v, sidx_a, didx_a, sidx_b, didx_b,
                   sidx_e, didx_e, rows_a, rows_b, acc_sh,
                   gsem_a, gsem_b, ssem_a, ssem_b):
        cid = lax.axis_index("c")
        sid = lax.axis_index("s")
        wid = cid * _NSUB + sid

        pltpu.async_copy(edges_hbm.at[wid], packed_v, gsem_b)
        for k in range(_RPT // _ZROWS):
            pltpu.async_copy(
                zeros_hbm, acc_sh.at[pl.ds(sid * _RPT + k * _ZROWS, _ZROWS)],
                gsem_a)

        @pl.when(sid == _NSUB - 1)
        def _():
            pltpu.sync_copy(zeros_hbm.at[pl.ds(0, 16)],
                            acc_sh.at[pl.ds(_NSUB * _RPT, 16)])

        for k in range(_RPT // _ZROWS):
            pltpu.make_async_copy(
                zeros_hbm, acc_sh.at[pl.ds(sid * _RPT + k * _ZROWS, _ZROWS)],
                gsem_a).wait()
        pltpu.make_async_copy(edges_hbm.at[wid], packed_v, gsem_b).wait()
        plsc.subcore_barrier()

        def unpack(j, sidx, didx):
            for c in range(_CHUNK // 16):
                v = packed_v[j, pl.ds(c * 16, 16)]
                sidx[0, pl.ds(c * 16, 16)] = v & 0xFFFF
                didx[0, pl.ds(c * 16, 16)] = lax.shift_right_logical(v, 16)

        unpack(0, sidx_a, didx_a)
        pltpu.async_copy(h_hbm.at[sidx_a.at[0]], rows_a, gsem_a)

        def body(i, carry):
            c0 = 2 * i
            c1 = c0 + 1
            unpack(c1, sidx_b, didx_b)
            pltpu.async_copy(h_hbm.at[sidx_b.at[0]], rows_b, gsem_b)
            pltpu.make_async_copy(h_hbm.at[sidx_a.at[0]], rows_a,
                                  gsem_a).wait()
            pltpu.async_copy(rows_a, acc_sh.at[didx_a.at[0]], ssem_a,
                             add=True)
            pltpu.make_async_copy(rows_a, acc_sh.at[didx_a.at[0]],
                                  ssem_a).wait()
            pltpu.make_async_copy(h_hbm.at[sidx_b.at[0]], rows_b,
                                  gsem_b).wait()
            pltpu.async_copy(rows_b, acc_sh.at[didx_b.at[0]], ssem_b,
                             add=True)

            @pl.when(i < _HALF - 1)
            def _():
                unpack(c0 + 2, sidx_a, didx_a)
                pltpu.async_copy(h_hbm.at[sidx_a.at[0]], rows_a, gsem_a)

            pltpu.make_async_copy(rows_b, acc_sh.at[didx_b.at[0]],
                                  ssem_b).wait()
            return carry

        lax.fori_loop(0, _HALF, body, 0)
        v = packed_v[_NCH - 1, pl.ds(0, _TAIL)]
        sidx_e[0, pl.ds(0, _TAIL)] = v & 0xFFFF
        didx_e[0, pl.ds(0, _TAIL)] = lax.shift_right_logical(v, 16)
        pltpu.async_copy(h_hbm.at[sidx_e.at[0]], rows_b.at[pl.ds(0, _TAIL)],
                         gsem_b)
        pltpu.make_async_copy(h_hbm.at[sidx_e.at[0]],
                              rows_b.at[pl.ds(0, _TAIL)], gsem_b).wait()
        pltpu.sync_copy(rows_b.at[pl.ds(0, _TAIL)],
                        acc_sh.at[didx_e.at[0]], add=True)
        plsc.subcore_barrier()

        @pl.when(cid == 0)
        def _():
            pltpu.sync_copy(acc_sh.at[pl.ds(sid * _RPT, _RPT)],
                            agg0_hbm.at[pl.ds(sid * _RPT, _RPT)])

            @pl.when(sid == _NSUB - 1)
            def _():
                pltpu.sync_copy(acc_sh.at[pl.ds(_NSUB * _RPT, 16)],
                                agg0_hbm.at[pl.ds(_NSUB * _RPT, 16)])

        @pl.when(cid == 1)
        def _():
            pltpu.sync_copy(acc_sh.at[pl.ds(sid * _RPT, _RPT)],
                            agg1_hbm.at[pl.ds(sid * _RPT, _RPT)])

            @pl.when(sid == _NSUB - 1)
            def _():
                pltpu.sync_copy(acc_sh.at[pl.ds(_NSUB * _RPT, 16)],
                                agg1_hbm.at[pl.ds(_NSUB * _RPT, 16)])

    return _sc_segsum


_BLK = 2000
_G = _N // _BLK


def _phase0(h_ref, a0_ref, a1_ref, w1t_ref, b1_ref, y_scr, st_scr, i):
    z = h_ref[...] + a0_ref[...] + a1_ref[...]
    y = jnp.dot(z, w1t_ref[...], preferred_element_type=jnp.float32)
    y = jnp.maximum(y + b1_ref[...], 0.0)
    y_scr[pl.ds(i * _BLK, _BLK), :] = y

    @pl.when(i == 0)
    def _():
        st_scr[...] = jnp.zeros_like(st_scr)

    s = jnp.sum(y, axis=0, keepdims=True)
    q = jnp.sum(y * y, axis=0, keepdims=True)
    st_scr[...] += jnp.concatenate(
        [s, q, jnp.zeros((6, _D), jnp.float32)], axis=0)


def _phase1_norm(g_ref, be_ref, w2t_ref, b2_ref, y_scr, st_scr, j):
    st = st_scr[...]
    mean = st[0:1, :] * (1.0 / _N)
    var = st[1:2, :] * (1.0 / _N) - mean * mean
    scale = g_ref[...] * lax.rsqrt(var + 1e-5)
    shift = be_ref[...] - mean * scale
    yn = y_scr[pl.ds(j * _BLK, _BLK), :] * scale + shift
    o = jnp.dot(yn, w2t_ref[...], preferred_element_type=jnp.float32)
    return jnp.maximum(o + b2_ref[...], 0.0)


def _layer_body(h_ref, a0_ref, a1_ref, w1t_ref, b1_ref, g_ref, be_ref,
                w2t_ref, b2_ref, o_ref, y_scr, st_scr):
    i = pl.program_id(0)

    @pl.when(i < _G)
    def _():
        _phase0(h_ref, a0_ref, a1_ref, w1t_ref, b1_ref, y_scr, st_scr, i)

    @pl.when(i >= _G)
    def _():
        o_ref[...] = _phase1_norm(g_ref, be_ref, w2t_ref, b2_ref,
                                  y_scr, st_scr, i - _G)


def _layer_res_body(h_ref, a0_ref, a1_ref, w1t_ref, b1_ref, g_ref, be_ref,
                    w2t_ref, b2_ref, r_ref, o_ref, y_scr, st_scr):
    i = pl.program_id(0)

    @pl.when(i < _G)
    def _():
        _phase0(h_ref, a0_ref, a1_ref, w1t_ref, b1_ref, y_scr, st_scr, i)

    @pl.when(i >= _G)
    def _():
        o_ref[...] = (_phase1_norm(g_ref, be_ref, w2t_ref, b2_ref,
                                   y_scr, st_scr, i - _G) + r_ref[...])


def _layer_head_body(h_ref, a0_ref, a1_ref, w1t_ref, b1_ref, g_ref, be_ref,
                     w2t_ref, b2_ref, r_ref, f1t_ref, f1b_ref, f2t_ref,
                     f2b_ref, o_ref, y_scr, st_scr):
    i = pl.program_id(0)

    @pl.when(i < _G)
    def _():
        _phase0(h_ref, a0_ref, a1_ref, w1t_ref, b1_ref, y_scr, st_scr, i)

    @pl.when(i >= _G)
    def _():
        h5 = (_phase1_norm(g_ref, be_ref, w2t_ref, b2_ref,
                           y_scr, st_scr, i - _G) + r_ref[...])
        t = jnp.dot(h5, f1t_ref[...], preferred_element_type=jnp.float32)
        t = h5 + jnp.maximum(t + f1b_ref[...], 0.0)
        o = jnp.dot(t, f2t_ref[...], preferred_element_type=jnp.float32)
        o_ref[...] = o + f2b_ref[...]


def _p0_map(i):
    return (jnp.minimum(i, _G - 1), 0)


def _p1_map(i):
    return (jnp.maximum(i - _G, 0), 0)


def _const_map(i):
    return (0, 0)


_layer_specs = [
    pl.BlockSpec((_BLK, _D), _p0_map),
    pl.BlockSpec((_BLK, _D), _p0_map),
    pl.BlockSpec((_BLK, _D), _p0_map),
    pl.BlockSpec((_D, _D), _const_map),
    pl.BlockSpec((1, _D), _const_map),
    pl.BlockSpec((1, _D), _const_map),
    pl.BlockSpec((1, _D), _const_map),
    pl.BlockSpec((_D, _D), _const_map),
    pl.BlockSpec((1, _D), _const_map),
]

_layer_scratch = [
    pltpu.VMEM((_N, _D), jnp.float32),
    pltpu.VMEM((8, _D), jnp.float32),
]

_layer = pl.pallas_call(
    _layer_body,
    grid=(2 * _G,),
    in_specs=_layer_specs,
    out_specs=pl.BlockSpec((_BLK, _D), _p1_map),
    out_shape=jax.ShapeDtypeStruct((_N, _D), jnp.float32),
    scratch_shapes=_layer_scratch,
)

_layer_res = pl.pallas_call(
    _layer_res_body,
    grid=(2 * _G,),
    in_specs=_layer_specs + [pl.BlockSpec((_BLK, _D), _p1_map)],
    out_specs=pl.BlockSpec((_BLK, _D), _p1_map),
    out_shape=jax.ShapeDtypeStruct((_N, _D), jnp.float32),
    scratch_shapes=_layer_scratch,
)

_layer_head = pl.pallas_call(
    _layer_head_body,
    grid=(2 * _G,),
    in_specs=_layer_specs + [
        pl.BlockSpec((_BLK, _D), _p1_map),
        pl.BlockSpec((_D, _D), _const_map),
        pl.BlockSpec((1, _D), _const_map),
        pl.BlockSpec((_D, 1), _const_map),
        pl.BlockSpec((1, 1), _const_map),
    ],
    out_specs=pl.BlockSpec((_BLK, 1), _p1_map),
    out_shape=jax.ShapeDtypeStruct((_N, 1), jnp.float32),
    scratch_shapes=_layer_scratch,
)


def kernel(x, edge_index, W1s, b1s, gammas, betas, W2s, b2s, fc1_w, fc1_b,
           fc2_w, fc2_b):
    packed = (edge_index[0] | (edge_index[1] << 16)).reshape(_WORKERS, _EPW)
    packed = jnp.pad(packed, ((0, 0), (0, _NCH * _CHUNK - _EPW)))
    packed = packed.reshape(_WORKERS, _NCH, _CHUNK)
    zeros = jnp.zeros((_ZROWS, _D), jnp.float32)
    W1ts = jnp.swapaxes(W1s, 1, 2)
    W2ts = jnp.swapaxes(W2s, 1, 2)

    sc_segsum = _get_sc_segsum()
    h = x
    x0 = x
    for i in range(5):
        agg0, agg1 = sc_segsum(h, packed, zeros)
        args = (h, agg0, agg1, W1ts[i], b1s[i][None], gammas[i][None],
                betas[i][None], W2ts[i], b2s[i][None])
        if i % 2 == 1:
            h = _layer_res(*args, x0)
            x0 = h
        else:
            h = _layer(*args)
    agg0, agg1 = sc_segsum(h, packed, zeros)
    return _layer_head(h, agg0, agg1, W1ts[5], b1s[5][None], gammas[5][None],
                       betas[5][None], W2ts[5], b2s[5][None], x0,
                       fc1_w.T, fc1_b[None], fc2_w.T, fc2_b[None])
